# baseline (speedup 1.0000x reference)
import functools
import jax
import jax.numpy as jnp
from jax import lax
from jax.experimental import pallas as pl
from jax.experimental.pallas import tpu as pltpu
from jax.experimental.pallas import tpu_sc as plsc

N = 10000
E = 320000
H = 128
NRBF = 20

NC = 2
NS = 16
NW = NC * NS
EPW = E // NW
CH = 40
NCHUNK = EPW // CH
NPAIR = NCHUNK // 2
NP = 10240
RPT = NP // NS

_mesh = plsc.VectorSubcoreMesh(core_axis_name="c", subcore_axis_name="s")


def _silu(x):
    return x * jax.nn.sigmoid(x)


def _pack_bf16(f, rows):
    b = f.astype(jnp.bfloat16)
    u = lax.bitcast_convert_type(b, jnp.uint16).astype(jnp.int32)
    u = u.reshape(rows, H // 32, 2, 16)
    ua = u[:, :, 0, :].reshape(rows, H // 2)
    ub = u[:, :, 1, :].reshape(rows, H // 2)
    return ua | (ub << 16)


def _bf16pair(v):
    shift = jnp.full((16,), 16, jnp.int32)
    mask = jnp.full((16,), -65536, jnp.int32)
    lo = lax.bitcast_convert_type(lax.shift_left(v, shift), jnp.float32)
    hi = lax.bitcast_convert_type(lax.bitwise_and(v, mask), jnp.float32)
    return lo, hi


BE = 2000

def _filters_body(rbf_ref, cut_ref, uv_ref, Wf1_ref, bf1_ref, Wf2_ref, bf2_ref,
                  fq_ref, fv0_ref, fv1_ref, fv2_ref):
    h = jnp.dot(rbf_ref[...], Wf1_ref[...],
                preferred_element_type=jnp.float32) + bf1_ref[...]
    h = _silu(h)
    f = jnp.dot(h, Wf2_ref[...],
                preferred_element_type=jnp.float32) + bf2_ref[...]
    cut = cut_ref[...]
    fq_ref[...] = f[:, :H] * cut
    fr = f[:, H:2 * H] * cut
    fmp = _pack_bf16(f[:, 2 * H:] * cut, BE)
    uv = uv_ref[...]
    fv0_ref[...] = jnp.concatenate(
        [_pack_bf16(fr * uv[:, 0:1], BE), fmp], axis=1)
    fv1_ref[...] = jnp.concatenate(
        [_pack_bf16(fr * uv[:, 1:2], BE), fmp], axis=1)
    fv2_ref[...] = jnp.concatenate(
        [_pack_bf16(fr * uv[:, 2:3], BE), fmp], axis=1)


def _filters_call(rbf, cut2, uv, Wf1, bf1, Wf2, bf2):
    grid = (E // BE,)
    eb = lambda i: (i, 0)
    wb = lambda i: (0, 0)
    return pl.pallas_call(
        _filters_body,
        grid=grid,
        in_specs=[
            pl.BlockSpec((BE, NRBF), eb),
            pl.BlockSpec((BE, 1), eb),
            pl.BlockSpec((BE, 3), eb),
            pl.BlockSpec((NRBF, H), wb),
            pl.BlockSpec((H,), lambda i: (0,)),
            pl.BlockSpec((H, 3 * H), wb),
            pl.BlockSpec((3 * H,), lambda i: (0,)),
        ],
        out_specs=[
            pl.BlockSpec((BE, H), eb),
            pl.BlockSpec((BE, H), eb),
            pl.BlockSpec((BE, H), eb),
            pl.BlockSpec((BE, H), eb),
        ],
        out_shape=[
            jax.ShapeDtypeStruct((E, H), jnp.float32),
            jax.ShapeDtypeStruct((E, H), jnp.int32),
            jax.ShapeDtypeStruct((E, H), jnp.int32),
            jax.ShapeDtypeStruct((E, H), jnp.int32),
        ],
    )(rbf, cut2, uv, Wf1, bf1, Wf2, bf2)


BNX = 2000

def _nodemlp_body(q_ref, W1_ref, b1_ref, W2_ref, b2_ref, xq_ref, xrm_ref):
    h = jnp.dot(q_ref[...], W1_ref[...],
                preferred_element_type=jnp.float32) + b1_ref[...]
    h = _silu(h)
    x = jnp.dot(h, W2_ref[...],
                preferred_element_type=jnp.float32) + b2_ref[...]
    xq_ref[...] = x[:, :H]
    xrm_ref[...] = jnp.concatenate(
        [_pack_bf16(x[:, H:2 * H], BNX), _pack_bf16(x[:, 2 * H:], BNX)],
        axis=1)


def _nodemlp_call(q, W1, b1, W2, b2):
    grid = (N // BNX,)
    nb = lambda i: (i, 0)
    wb = lambda i: (0, 0)
    return pl.pallas_call(
        _nodemlp_body,
        grid=grid,
        in_specs=[
            pl.BlockSpec((BNX, H), nb),
            pl.BlockSpec((H, 3 * H), wb),
            pl.BlockSpec((3 * H,), lambda i: (0,)),
            pl.BlockSpec((3 * H, 3 * H), wb),
            pl.BlockSpec((3 * H,), lambda i: (0,)),
        ],
        out_specs=[
            pl.BlockSpec((BNX, H), nb),
            pl.BlockSpec((BNX, H), nb),
        ],
        out_shape=[
            jax.ShapeDtypeStruct((N, H), jnp.float32),
            jax.ShapeDtypeStruct((N, H), jnp.int32),
        ],
    )(q, W1, b1, W2, b2)


def _zero_and_drain_setup(buf, accum_sh, sid):
    zero16 = jnp.zeros((16,), jnp.float32)

    def zrow(r, _):
        for j in range(H // 16):
            buf[r, pl.ds(j * 16, 16)] = zero16
        return 0
    lax.fori_loop(0, CH, zrow, 0)
    for k in range(RPT // CH):
        pltpu.sync_copy(buf, accum_sh.at[pl.ds(sid * RPT + k * CH, CH)])


def _drain(buf, accum_sh, out_hbm, cid, sid):
    for k in range(RPT // CH):
        r0 = sid * RPT + k * CH
        pltpu.sync_copy(accum_sh.at[pl.ds(r0, CH)], buf)
        pltpu.sync_copy(buf, out_hbm.at[cid, pl.ds(r0, CH)])


@functools.partial(
    pl.kernel,
    mesh=_mesh,
    out_type=jax.ShapeDtypeStruct((NC, NP, H), jnp.float32),
    scratch_types=[
        pltpu.VMEM((CH,), jnp.int32),
        pltpu.VMEM((CH,), jnp.int32),
        pltpu.VMEM((CH, H), jnp.float32),
        pltpu.VMEM_SHARED((NP, H), jnp.float32),
        pltpu.SemaphoreType.DMA,
        pltpu.SemaphoreType.DMA,
    ],
)
def _sc_deg(tgt_hbm, out_hbm, tgtA, tgtB, ones_v, accum_sh, semA, semB):
    cid = lax.axis_index("c")
    sid = lax.axis_index("s")
    wid = sid * NC + cid
    one16 = jnp.ones((16,), jnp.float32)

    _zero_and_drain_setup(ones_v, accum_sh, sid)

    def orow(r, _):
        for j in range(H // 16):
            ones_v[r, pl.ds(j * 16, 16)] = one16
        return 0
    lax.fori_loop(0, CH, orow, 0)
    plsc.subcore_barrier()

    def base(j):
        return pl.multiple_of(wid * EPW + j * CH, 8)

    def fire(j, buf, sem):
        pltpu.async_copy(tgt_hbm.at[pl.ds(base(j), CH)], buf, sem)

    def drain(j, buf, sem):
        pltpu.make_async_copy(tgt_hbm.at[pl.ds(base(j), CH)], buf, sem).wait()

    fire(0, tgtA, semA)
    fire(1, tgtB, semB)

    def pair(k, _):
        a = 2 * k
        b = a + 1
        drain(a, tgtA, semA)
        pltpu.sync_copy(ones_v, accum_sh.at[tgtA], add=True)

        @pl.when(a + 2 < NCHUNK)
        def _():
            fire(a + 2, tgtA, semA)
        drain(b, tgtB, semB)
        pltpu.sync_copy(ones_v, accum_sh.at[tgtB], add=True)

        @pl.when(b + 2 < NCHUNK)
        def _():
            fire(b + 2, tgtB, semB)
        return 0
    lax.fori_loop(0, NPAIR, pair, 0)

    plsc.subcore_barrier()
    _drain(ones_v, accum_sh, out_hbm, cid, sid)


@functools.partial(
    pl.kernel,
    mesh=_mesh,
    out_type=jax.ShapeDtypeStruct((NC, NP, H), jnp.float32),
    scratch_types=[
        pltpu.VMEM((CH,), jnp.int32),
        pltpu.VMEM((CH,), jnp.int32),
        pltpu.VMEM((CH,), jnp.int32),
        pltpu.VMEM((CH,), jnp.int32),
        pltpu.VMEM((CH, H), jnp.float32),
        pltpu.VMEM((CH, H), jnp.float32),
        pltpu.VMEM((CH, H), jnp.float32),
        pltpu.VMEM((CH, H), jnp.float32),
        pltpu.VMEM_SHARED((NP, H), jnp.float32),
        pltpu.SemaphoreType.DMA,
        pltpu.SemaphoreType.DMA,
        pltpu.SemaphoreType.DMA,
        pltpu.SemaphoreType.DMA,
        pltpu.SemaphoreType.DMA,
        pltpu.SemaphoreType.DMA,
    ],
)
def _sc_scalar(src_hbm, tgt_hbm, xq_hbm, fq_hbm, out_hbm,
               srcA, srcB, tgtA, tgtB, xqA, xqB, fqA, fqB, accum_sh,
               semISA, semISB, semITA, semITB, semRA, semRB):
    cid = lax.axis_index("c")
    sid = lax.axis_index("s")
    wid = sid * NC + cid

    _zero_and_drain_setup(fqA, accum_sh, sid)
    plsc.subcore_barrier()

    def base(j):
        return pl.multiple_of(wid * EPW + j * CH, 8)

    def fire_src(j, buf, sem):
        pltpu.async_copy(src_hbm.at[pl.ds(base(j), CH)], buf, sem)

    def drain_src(j, buf, sem):
        pltpu.make_async_copy(src_hbm.at[pl.ds(base(j), CH)], buf, sem).wait()

    def fire_tgt(j, buf, sem):
        pltpu.async_copy(tgt_hbm.at[pl.ds(base(j), CH)], buf, sem)

    def drain_tgt(j, buf, sem):
        pltpu.make_async_copy(tgt_hbm.at[pl.ds(base(j), CH)], buf, sem).wait()

    def fire_rows(j, sbuf, xq, fq, sem):
        pltpu.async_copy(xq_hbm.at[sbuf], xq, sem)
        pltpu.async_copy(fq_hbm.at[pl.ds(base(j), CH)], fq, sem)

    def drain_rows(j, sbuf, xq, fq, sem):
        pltpu.make_async_copy(xq_hbm.at[sbuf], xq, sem).wait()
        pltpu.make_async_copy(fq_hbm.at[pl.ds(base(j), CH)], fq, sem).wait()

    def compute(xq, fq):
        def row(r, _):
            for j in range(H // 16):
                sl = pl.ds(j * 16, 16)
                xq[r, sl] = xq[r, sl] * fq[r, sl]
            return 0
        lax.fori_loop(0, CH, row, 0)

    fire_src(0, srcA, semISA)
    fire_tgt(0, tgtA, semITA)
    fire_src(1, srcB, semISB)
    fire_tgt(1, tgtB, semITB)
    drain_src(0, srcA, semISA)
    fire_rows(0, srcA, xqA, fqA, semRA)

    def pair(k, _):
        a = 2 * k
        b = a + 1
        a2 = a + 2
        b2 = b + 2
        drain_src(b, srcB, semISB)
        fire_rows(b, srcB, xqB, fqB, semRB)
        drain_rows(a, srcA, xqA, fqA, semRA)

        @pl.when(a2 < NCHUNK)
        def _():
            fire_src(a2, srcA, semISA)
        drain_tgt(a, tgtA, semITA)
        compute(xqA, fqA)
        pltpu.sync_copy(xqA, accum_sh.at[tgtA], add=True)

        @pl.when(a2 < NCHUNK)
        def _():
            fire_tgt(a2, tgtA, semITA)
            drain_src(a2, srcA, semISA)
            fire_rows(a2, srcA, xqA, fqA, semRA)
        drain_rows(b, srcB, xqB, fqB, semRB)
        drain_tgt(b, tgtB, semITB)
        compute(xqB, fqB)
        pltpu.sync_copy(xqB, accum_sh.at[tgtB], add=True)

        @pl.when(b2 < NCHUNK)
        def _():
            fire_src(b2, srcB, semISB)
            fire_tgt(b2, tgtB, semITB)
        return 0
    lax.fori_loop(0, NPAIR, pair, 0)

    plsc.subcore_barrier()
    _drain(fqA, accum_sh, out_hbm, cid, sid)


@functools.partial(
    pl.kernel,
    mesh=_mesh,
    out_type=jax.ShapeDtypeStruct((NC, NP, H), jnp.float32),
    scratch_types=[
        pltpu.VMEM((CH,), jnp.int32),
        pltpu.VMEM((CH,), jnp.int32),
        pltpu.VMEM((CH,), jnp.int32),
        pltpu.VMEM((CH,), jnp.int32),
        pltpu.VMEM((CH, H), jnp.int32),
        pltpu.VMEM((CH, H), jnp.int32),
        pltpu.VMEM((CH, H), jnp.float32),
        pltpu.VMEM((CH, H), jnp.float32),
        pltpu.VMEM((CH, H), jnp.int32),
        pltpu.VMEM((CH, H), jnp.int32),
        pltpu.VMEM((CH, H), jnp.float32),
        pltpu.VMEM_SHARED((NP, H), jnp.float32),
        pltpu.SemaphoreType.DMA,
        pltpu.SemaphoreType.DMA,
        pltpu.SemaphoreType.DMA,
        pltpu.SemaphoreType.DMA,
        pltpu.SemaphoreType.DMA,
        pltpu.SemaphoreType.DMA,
    ],
)
def _sc_vec(src_hbm, tgt_hbm, xrm_hbm, muc_hbm, fv_hbm, out_hbm,
            srcA, srcB, tgtA, tgtB, xrmA, xrmB, mcA, mcB, fvA, fvB, val,
            accum_sh, semISA, semISB, semITA, semITB, semRA, semRB):
    cid = lax.axis_index("c")
    sid = lax.axis_index("s")
    wid = sid * NC + cid

    _zero_and_drain_setup(val, accum_sh, sid)
    plsc.subcore_barrier()

    def base(j):
        return pl.multiple_of(wid * EPW + j * CH, 8)

    def fire_src(j, buf, sem):
        pltpu.async_copy(src_hbm.at[pl.ds(base(j), CH)], buf, sem)

    def drain_src(j, buf, sem):
        pltpu.make_async_copy(src_hbm.at[pl.ds(base(j), CH)], buf, sem).wait()

    def fire_tgt(j, buf, sem):
        pltpu.async_copy(tgt_hbm.at[pl.ds(base(j), CH)], buf, sem)

    def drain_tgt(j, buf, sem):
        pltpu.make_async_copy(tgt_hbm.at[pl.ds(base(j), CH)], buf, sem).wait()

    def fire_rows(j, sbuf, xrm, mc, fv, sem):
        pltpu.async_copy(xrm_hbm.at[sbuf], xrm, sem)
        pltpu.async_copy(muc_hbm.at[sbuf], mc, sem)
        pltpu.async_copy(fv_hbm.at[pl.ds(base(j), CH)], fv, sem)

    def drain_rows(j, sbuf, xrm, mc, fv, sem):
        pltpu.make_async_copy(xrm_hbm.at[sbuf], xrm, sem).wait()
        pltpu.make_async_copy(muc_hbm.at[sbuf], mc, sem).wait()
        pltpu.make_async_copy(fv_hbm.at[pl.ds(base(j), CH)], fv, sem).wait()

    def compute(xrm, mc, fv):
        def row(r, _):
            for m in range(H // 32):
                slr = pl.ds(m * 16, 16)
                slm = pl.ds(H // 2 + m * 16, 16)
                olo = pl.ds(32 * m, 16)
                ohi = pl.ds(32 * m + 16, 16)
                xrl, xrh = _bf16pair(xrm[r, slr])
                xml, xmh = _bf16pair(xrm[r, slm])
                frl, frh = _bf16pair(fv[r, slr])
                fml, fmh = _bf16pair(fv[r, slm])
                val[r, olo] = xrl * frl + mc[r, olo] * (xml * fml)
                val[r, ohi] = xrh * frh + mc[r, ohi] * (xmh * fmh)
            return 0
        lax.fori_loop(0, CH, row, 0)

    fire_src(0, srcA, semISA)
    fire_tgt(0, tgtA, semITA)
    fire_src(1, srcB, semISB)
    fire_tgt(1, tgtB, semITB)
    drain_src(0, srcA, semISA)
    fire_rows(0, srcA, xrmA, mcA, fvA, semRA)

    def pair(k, _):
        a = 2 * k
        b = a + 1
        a2 = a + 2
        b2 = b + 2
        drain_src(b, srcB, semISB)
        fire_rows(b, srcB, xrmB, mcB, fvB, semRB)
        drain_rows(a, srcA, xrmA, mcA, fvA, semRA)

        @pl.when(a2 < NCHUNK)
        def _():
            fire_src(a2, srcA, semISA)
        drain_tgt(a, tgtA, semITA)
        compute(xrmA, mcA, fvA)
        pltpu.sync_copy(val, accum_sh.at[tgtA], add=True)

        @pl.when(a2 < NCHUNK)
        def _():
            fire_tgt(a2, tgtA, semITA)
            drain_src(a2, srcA, semISA)
            fire_rows(a2, srcA, xrmA, mcA, fvA, semRA)
        drain_rows(b, srcB, xrmB, mcB, fvB, semRB)
        drain_tgt(b, tgtB, semITB)
        compute(xrmB, mcB, fvB)
        pltpu.sync_copy(val, accum_sh.at[tgtB], add=True)

        @pl.when(b2 < NCHUNK)
        def _():
            fire_src(b2, srcB, semISB)
            fire_tgt(b2, tgtB, semITB)
        return 0
    lax.fori_loop(0, NPAIR, pair, 0)

    plsc.subcore_barrier()
    _drain(val, accum_sh, out_hbm, cid, sid)


BNM = 1000

def _mix_body(q_ref, mu_ref, pd_ref, pA_ref, p0_ref, p1_ref, p2_ref,
              Wv_ref, Wm1_ref, bm1_ref, Wm2_ref, bm2_ref, qo_ref, muo_ref):
    q = q_ref[...]
    mu = mu_ref[...]
    deg = jnp.maximum(pd_ref[0, :, :1] + pd_ref[1, :, :1], 1.0)
    q1 = q + (pA_ref[0] + pA_ref[1]) / deg
    vm = jnp.stack([p0_ref[0] + p0_ref[1],
                    p1_ref[0] + p1_ref[1],
                    p2_ref[0] + p2_ref[1]], axis=1)
    mu1 = mu + vm / deg[:, :, None]
    mc = jnp.dot(mu1.reshape(BNM * 3, H), Wv_ref[...],
                 preferred_element_type=jnp.float32).reshape(BNM, 3, 2 * H)
    mu_v = mc[..., :H]
    mu_w = mc[..., H:]
    mu_v_norm = jnp.sqrt(jnp.sum(mu_v * mu_v, axis=1) + 1e-8)
    si = jnp.concatenate([q1, mu_v_norm], axis=-1)
    h = jnp.dot(si, Wm1_ref[...],
                preferred_element_type=jnp.float32) + bm1_ref[...]
    h = _silu(h)
    delta = jnp.dot(h, Wm2_ref[...],
                    preferred_element_type=jnp.float32) + bm2_ref[...]
    dq = delta[:, :H]
    dsc = delta[:, H:2 * H]
    dqmu = delta[:, 2 * H:]
    inner = jnp.sum(mu_v * mu_w, axis=1)
    qo_ref[...] = q1 + dq + dqmu * inner
    muo_ref[...] = mu1 + mu_w * dsc[:, None, :]


def _mix_call(q, mu, pd, pA, p0, p1, p2, Wv, Wm1, bm1, Wm2, bm2):
    grid = (N // BNM,)
    nb = lambda i: (i, 0)
    wb = lambda i: (0, 0)
    nb3 = lambda i: (i, 0, 0)
    pb = lambda i: (0, i, 0)
    return pl.pallas_call(
        _mix_body,
        grid=grid,
        in_specs=[
            pl.BlockSpec((BNM, H), nb),
            pl.BlockSpec((BNM, 3, H), nb3),
            pl.BlockSpec((NC, BNM, H), pb),
            pl.BlockSpec((NC, BNM, H), pb),
            pl.BlockSpec((NC, BNM, H), pb),
            pl.BlockSpec((NC, BNM, H), pb),
            pl.BlockSpec((NC, BNM, H), pb),
            pl.BlockSpec((H, 2 * H), wb),
            pl.BlockSpec((2 * H, 3 * H), wb),
            pl.BlockSpec((3 * H,), lambda i: (0,)),
            pl.BlockSpec((3 * H, 3 * H), wb),
            pl.BlockSpec((3 * H,), lambda i: (0,)),
        ],
        out_specs=[
            pl.BlockSpec((BNM, H), nb),
            pl.BlockSpec((BNM, 3, H), nb3),
        ],
        out_shape=[
            jax.ShapeDtypeStruct((N, H), jnp.float32),
            jax.ShapeDtypeStruct((N, 3, H), jnp.float32),
        ],
    )(q, mu, pd, pA, p0, p1, p2, Wv, Wm1, bm1, Wm2, bm2)


def kernel(q, mu, edge_index, rbf, unit_vectors, cutoff_values,
           W1, b1, W2, b2, Wf1, bf1, Wf2, bf2, Wv, Wm1, bm1, Wm2, bm2):
    src = edge_index[1]
    tgt = edge_index[0]
    cut2 = cutoff_values[:, None]
    fq, fv0, fv1, fv2 = _filters_call(rbf, cut2, unit_vectors,
                                      Wf1, bf1, Wf2, bf2)
    xq, xrm = _nodemlp_call(q, W1, b1, W2, b2)
    mu0 = mu[:, 0]
    mu1t = mu[:, 1]
    mu2t = mu[:, 2]
    pd = _sc_deg(tgt)
    pA = _sc_scalar(src, tgt, xq, fq)
    p0 = _sc_vec(src, tgt, xrm, mu0, fv0)
    p1 = _sc_vec(src, tgt, xrm, mu1t, fv1)
    p2 = _sc_vec(src, tgt, xrm, mu2t, fv2)
    return _mix_call(q, mu, pd, pA, p0, p1, p2, Wv, Wm1, bm1, Wm2, bm2)


# speedup vs baseline: 2.7273x; 2.7273x over previous
import functools
import jax
import jax.numpy as jnp
from jax import lax
from jax.experimental import pallas as pl
from jax.experimental.pallas import tpu as pltpu
from jax.experimental.pallas import tpu_sc as plsc

N = 10000
E = 320000
H = 128
NRBF = 20

NC = 2
NS = 16
NW = NC * NS
EPW = E // NW
CH = 40
NCHUNK = EPW // CH
NPAIR = NCHUNK // 2
NP = 10240
RPT = NP // NS

_mesh = plsc.VectorSubcoreMesh(core_axis_name="c", subcore_axis_name="s")


def _silu(x):
    return x * jax.nn.sigmoid(x)


BE = 2000

def _filters_body(rbf_ref, cut_ref, uv_ref, Wf1_ref, bf1_ref, Wf2_ref, bf2_ref,
                  fq_ref, fr0_ref, fr1_ref, fr2_ref, fmu_ref):
    h = jnp.dot(rbf_ref[...], Wf1_ref[...],
                preferred_element_type=jnp.float32) + bf1_ref[...]
    h = _silu(h)
    f = jnp.dot(h, Wf2_ref[...],
                preferred_element_type=jnp.float32) + bf2_ref[...]
    cut = cut_ref[...]
    fq_ref[...] = f[:, :H] * cut
    fmu_ref[...] = f[:, 2 * H:] * cut
    fr = f[:, H:2 * H] * cut
    uv = uv_ref[...]
    fr0_ref[...] = fr * uv[:, 0:1]
    fr1_ref[...] = fr * uv[:, 1:2]
    fr2_ref[...] = fr * uv[:, 2:3]


def _filters_call(rbf, cut2, uv, Wf1, bf1, Wf2, bf2):
    grid = (E // BE,)
    eb = lambda i: (i, 0)
    wb = lambda i: (0, 0)
    return pl.pallas_call(
        _filters_body,
        grid=grid,
        in_specs=[
            pl.BlockSpec((BE, NRBF), eb),
            pl.BlockSpec((BE, 1), eb),
            pl.BlockSpec((BE, 3), eb),
            pl.BlockSpec((NRBF, H), wb),
            pl.BlockSpec((H,), lambda i: (0,)),
            pl.BlockSpec((H, 3 * H), wb),
            pl.BlockSpec((3 * H,), lambda i: (0,)),
        ],
        out_specs=[pl.BlockSpec((BE, H), eb)] * 5,
        out_shape=[jax.ShapeDtypeStruct((E, H), jnp.float32)] * 5,
    )(rbf, cut2, uv, Wf1, bf1, Wf2, bf2)


BNX = 2000

def _nodemlp_body(q_ref, W1_ref, b1_ref, W2_ref, b2_ref,
                  xq_ref, xr_ref, xm_ref):
    h = jnp.dot(q_ref[...], W1_ref[...],
                preferred_element_type=jnp.float32) + b1_ref[...]
    h = _silu(h)
    x = jnp.dot(h, W2_ref[...],
                preferred_element_type=jnp.float32) + b2_ref[...]
    xq_ref[...] = x[:, :H]
    xr_ref[...] = x[:, H:2 * H]
    xm_ref[...] = x[:, 2 * H:]


def _nodemlp_call(q, W1, b1, W2, b2):
    grid = (N // BNX,)
    nb = lambda i: (i, 0)
    wb = lambda i: (0, 0)
    return pl.pallas_call(
        _nodemlp_body,
        grid=grid,
        in_specs=[
            pl.BlockSpec((BNX, H), nb),
            pl.BlockSpec((H, 3 * H), wb),
            pl.BlockSpec((3 * H,), lambda i: (0,)),
            pl.BlockSpec((3 * H, 3 * H), wb),
            pl.BlockSpec((3 * H,), lambda i: (0,)),
        ],
        out_specs=[pl.BlockSpec((BNX, H), nb)] * 3,
        out_shape=[jax.ShapeDtypeStruct((N, H), jnp.float32)] * 3,
    )(q, W1, b1, W2, b2)


def _zero_and_drain_setup(buf, accum_sh, sid):
    zero16 = jnp.zeros((16,), jnp.float32)

    def zrow(r, _):
        for j in range(H // 16):
            buf[r, pl.ds(j * 16, 16)] = zero16
        return 0
    lax.fori_loop(0, CH, zrow, 0)
    for k in range(RPT // CH):
        pltpu.sync_copy(buf, accum_sh.at[pl.ds(sid * RPT + k * CH, CH)])


def _drain(buf, accum_sh, out_hbm, cid, sid):
    for k in range(RPT // CH):
        r0 = sid * RPT + k * CH
        pltpu.sync_copy(accum_sh.at[pl.ds(r0, CH)], buf)
        pltpu.sync_copy(buf, out_hbm.at[cid, pl.ds(r0, CH)])


@functools.partial(
    pl.kernel,
    mesh=_mesh,
    out_type=jax.ShapeDtypeStruct((NC, NP, H), jnp.float32),
    scratch_types=[
        pltpu.VMEM((CH,), jnp.int32),
        pltpu.VMEM((CH,), jnp.int32),
        pltpu.VMEM((CH, H), jnp.float32),
        pltpu.VMEM_SHARED((NP, H), jnp.float32),
        pltpu.SemaphoreType.DMA,
        pltpu.SemaphoreType.DMA,
    ],
)
def _sc_deg(tgt_hbm, out_hbm, tgtA, tgtB, ones_v, accum_sh, semA, semB):
    cid = lax.axis_index("c")
    sid = lax.axis_index("s")
    wid = sid * NC + cid
    one16 = jnp.ones((16,), jnp.float32)

    _zero_and_drain_setup(ones_v, accum_sh, sid)

    def orow(r, _):
        for j in range(H // 16):
            ones_v[r, pl.ds(j * 16, 16)] = one16
        return 0
    lax.fori_loop(0, CH, orow, 0)
    plsc.subcore_barrier()

    def base(j):
        return pl.multiple_of(wid * EPW + j * CH, 8)

    def fire(j, buf, sem):
        pltpu.async_copy(tgt_hbm.at[pl.ds(base(j), CH)], buf, sem)

    def drain(j, buf, sem):
        pltpu.make_async_copy(tgt_hbm.at[pl.ds(base(j), CH)], buf, sem).wait()

    fire(0, tgtA, semA)
    fire(1, tgtB, semB)

    def pair(k, _):
        a = 2 * k
        b = a + 1
        drain(a, tgtA, semA)
        pltpu.sync_copy(ones_v, accum_sh.at[tgtA], add=True)

        @pl.when(a + 2 < NCHUNK)
        def _():
            fire(a + 2, tgtA, semA)
        drain(b, tgtB, semB)
        pltpu.sync_copy(ones_v, accum_sh.at[tgtB], add=True)

        @pl.when(b + 2 < NCHUNK)
        def _():
            fire(b + 2, tgtB, semB)
        return 0
    lax.fori_loop(0, NPAIR, pair, 0)

    plsc.subcore_barrier()
    _drain(ones_v, accum_sh, out_hbm, cid, sid)


@functools.partial(
    pl.kernel,
    mesh=_mesh,
    out_type=jax.ShapeDtypeStruct((NC, NP, H), jnp.float32),
    scratch_types=[
        pltpu.VMEM((CH,), jnp.int32),
        pltpu.VMEM((CH,), jnp.int32),
        pltpu.VMEM((CH,), jnp.int32),
        pltpu.VMEM((CH,), jnp.int32),
        pltpu.VMEM((CH, H), jnp.float32),
        pltpu.VMEM((CH, H), jnp.float32),
        pltpu.VMEM((CH, H), jnp.float32),
        pltpu.VMEM((CH, H), jnp.float32),
        pltpu.VMEM_SHARED((NP, H), jnp.float32),
        pltpu.SemaphoreType.DMA,
        pltpu.SemaphoreType.DMA,
        pltpu.SemaphoreType.DMA,
        pltpu.SemaphoreType.DMA,
        pltpu.SemaphoreType.DMA,
        pltpu.SemaphoreType.DMA,
    ],
)
def _sc_scalar(src_hbm, tgt_hbm, xq_hbm, fq_hbm, out_hbm,
               srcA, srcB, tgtA, tgtB, xqA, xqB, fqA, fqB, accum_sh,
               semISA, semISB, semITA, semITB, semRA, semRB):
    cid = lax.axis_index("c")
    sid = lax.axis_index("s")
    wid = sid * NC + cid

    _zero_and_drain_setup(fqA, accum_sh, sid)
    plsc.subcore_barrier()

    def base(j):
        return pl.multiple_of(wid * EPW + j * CH, 8)

    def fire_src(j, buf, sem):
        pltpu.async_copy(src_hbm.at[pl.ds(base(j), CH)], buf, sem)

    def drain_src(j, buf, sem):
        pltpu.make_async_copy(src_hbm.at[pl.ds(base(j), CH)], buf, sem).wait()

    def fire_tgt(j, buf, sem):
        pltpu.async_copy(tgt_hbm.at[pl.ds(base(j), CH)], buf, sem)

    def drain_tgt(j, buf, sem):
        pltpu.make_async_copy(tgt_hbm.at[pl.ds(base(j), CH)], buf, sem).wait()

    def fire_rows(j, sbuf, xq, fq, sem):
        pltpu.async_copy(xq_hbm.at[sbuf], xq, sem)
        pltpu.async_copy(fq_hbm.at[pl.ds(base(j), CH)], fq, sem)

    def drain_rows(j, sbuf, xq, fq, sem):
        pltpu.make_async_copy(xq_hbm.at[sbuf], xq, sem).wait()
        pltpu.make_async_copy(fq_hbm.at[pl.ds(base(j), CH)], fq, sem).wait()

    def compute(xq, fq):
        def row(r, _):
            for j in range(H // 16):
                sl = pl.ds(j * 16, 16)
                xq[r, sl] = xq[r, sl] * fq[r, sl]
            return 0
        lax.fori_loop(0, CH, row, 0)

    fire_src(0, srcA, semISA)
    fire_tgt(0, tgtA, semITA)
    fire_src(1, srcB, semISB)
    fire_tgt(1, tgtB, semITB)
    drain_src(0, srcA, semISA)
    fire_rows(0, srcA, xqA, fqA, semRA)

    def pair(k, _):
        a = 2 * k
        b = a + 1
        a2 = a + 2
        b2 = b + 2
        drain_src(b, srcB, semISB)
        fire_rows(b, srcB, xqB, fqB, semRB)
        drain_rows(a, srcA, xqA, fqA, semRA)

        @pl.when(a2 < NCHUNK)
        def _():
            fire_src(a2, srcA, semISA)
        drain_tgt(a, tgtA, semITA)
        compute(xqA, fqA)
        pltpu.sync_copy(xqA, accum_sh.at[tgtA], add=True)

        @pl.when(a2 < NCHUNK)
        def _():
            fire_tgt(a2, tgtA, semITA)
            drain_src(a2, srcA, semISA)
            fire_rows(a2, srcA, xqA, fqA, semRA)
        drain_rows(b, srcB, xqB, fqB, semRB)
        drain_tgt(b, tgtB, semITB)
        compute(xqB, fqB)
        pltpu.sync_copy(xqB, accum_sh.at[tgtB], add=True)

        @pl.when(b2 < NCHUNK)
        def _():
            fire_src(b2, srcB, semISB)
            fire_tgt(b2, tgtB, semITB)
        return 0
    lax.fori_loop(0, NPAIR, pair, 0)

    plsc.subcore_barrier()
    _drain(fqA, accum_sh, out_hbm, cid, sid)


@functools.partial(
    pl.kernel,
    mesh=_mesh,
    out_type=jax.ShapeDtypeStruct((NC, NP, H), jnp.float32),
    scratch_types=[
        pltpu.VMEM((CH,), jnp.int32),
        pltpu.VMEM((CH,), jnp.int32),
        pltpu.VMEM((CH,), jnp.int32),
        pltpu.VMEM((CH,), jnp.int32),
        pltpu.VMEM((CH, H), jnp.float32),
        pltpu.VMEM((CH, H), jnp.float32),
        pltpu.VMEM((CH, H), jnp.float32),
        pltpu.VMEM((CH, H), jnp.float32),
        pltpu.VMEM((CH, H), jnp.float32),
        pltpu.VMEM((CH, H), jnp.float32),
        pltpu.VMEM((CH, H), jnp.float32),
        pltpu.VMEM((CH, H), jnp.float32),
        pltpu.VMEM((CH, H), jnp.float32),
        pltpu.VMEM_SHARED((NP, H), jnp.float32),
        pltpu.SemaphoreType.DMA,
        pltpu.SemaphoreType.DMA,
        pltpu.SemaphoreType.DMA,
        pltpu.SemaphoreType.DMA,
        pltpu.SemaphoreType.DMA,
        pltpu.SemaphoreType.DMA,
        pltpu.SemaphoreType.DMA,
    ],
)
def _sc_vec(src_hbm, tgt_hbm, xr_hbm, xm_hbm, muc_hbm, frc_hbm, fmu_hbm,
            out_hbm, srcA, srcB, tgtA, tgtB, xrA, xrB, xmA, xmB, mcA, mcB,
            frA, frB, fm, accum_sh,
            semISA, semISB, semITA, semITB, semRA, semRB, semF):
    cid = lax.axis_index("c")
    sid = lax.axis_index("s")
    wid = sid * NC + cid

    _zero_and_drain_setup(frA, accum_sh, sid)
    plsc.subcore_barrier()

    def base(j):
        return pl.multiple_of(wid * EPW + j * CH, 8)

    def fire_src(j, buf, sem):
        pltpu.async_copy(src_hbm.at[pl.ds(base(j), CH)], buf, sem)

    def drain_src(j, buf, sem):
        pltpu.make_async_copy(src_hbm.at[pl.ds(base(j), CH)], buf, sem).wait()

    def fire_tgt(j, buf, sem):
        pltpu.async_copy(tgt_hbm.at[pl.ds(base(j), CH)], buf, sem)

    def drain_tgt(j, buf, sem):
        pltpu.make_async_copy(tgt_hbm.at[pl.ds(base(j), CH)], buf, sem).wait()

    def fire_rows(j, sbuf, xr, xm, mc, fr, sem):
        pltpu.async_copy(xr_hbm.at[sbuf], xr, sem)
        pltpu.async_copy(xm_hbm.at[sbuf], xm, sem)
        pltpu.async_copy(muc_hbm.at[sbuf], mc, sem)
        pltpu.async_copy(frc_hbm.at[pl.ds(base(j), CH)], fr, sem)

    def drain_rows(j, sbuf, xr, xm, mc, fr, sem):
        pltpu.make_async_copy(xr_hbm.at[sbuf], xr, sem).wait()
        pltpu.make_async_copy(xm_hbm.at[sbuf], xm, sem).wait()
        pltpu.make_async_copy(muc_hbm.at[sbuf], mc, sem).wait()
        pltpu.make_async_copy(frc_hbm.at[pl.ds(base(j), CH)], fr, sem).wait()

    def fire_fm(j):
        pltpu.async_copy(fmu_hbm.at[pl.ds(base(j), CH)], fm, semF)

    def drain_fm(j):
        pltpu.make_async_copy(fmu_hbm.at[pl.ds(base(j), CH)], fm, semF).wait()

    def compute(xr, xm, mc, fr):
        def row(r, _):
            for j in range(H // 16):
                sl = pl.ds(j * 16, 16)
                xr[r, sl] = (xr[r, sl] * fr[r, sl]
                             + mc[r, sl] * (xm[r, sl] * fm[r, sl]))
            return 0
        lax.fori_loop(0, CH, row, 0)

    fire_src(0, srcA, semISA)
    fire_tgt(0, tgtA, semITA)
    fire_src(1, srcB, semISB)
    fire_tgt(1, tgtB, semITB)
    drain_src(0, srcA, semISA)
    fire_rows(0, srcA, xrA, xmA, mcA, frA, semRA)
    fire_fm(0)

    def pair(k, _):
        a = 2 * k
        b = a + 1
        a2 = a + 2
        b2 = b + 2
        drain_src(b, srcB, semISB)
        fire_rows(b, srcB, xrB, xmB, mcB, frB, semRB)
        drain_rows(a, srcA, xrA, xmA, mcA, frA, semRA)

        @pl.when(a2 < NCHUNK)
        def _():
            fire_src(a2, srcA, semISA)
        drain_tgt(a, tgtA, semITA)
        drain_fm(a)
        compute(xrA, xmA, mcA, frA)
        fire_fm(b)
        pltpu.sync_copy(xrA, accum_sh.at[tgtA], add=True)

        @pl.when(a2 < NCHUNK)
        def _():
            fire_tgt(a2, tgtA, semITA)
            drain_src(a2, srcA, semISA)
            fire_rows(a2, srcA, xrA, xmA, mcA, frA, semRA)
        drain_rows(b, srcB, xrB, xmB, mcB, frB, semRB)
        drain_tgt(b, tgtB, semITB)
        drain_fm(b)
        compute(xrB, xmB, mcB, frB)

        @pl.when(a2 < NCHUNK)
        def _():
            fire_fm(a2)
        pltpu.sync_copy(xrB, accum_sh.at[tgtB], add=True)

        @pl.when(b2 < NCHUNK)
        def _():
            fire_src(b2, srcB, semISB)
            fire_tgt(b2, tgtB, semITB)
        return 0
    lax.fori_loop(0, NPAIR, pair, 0)

    plsc.subcore_barrier()
    _drain(frA, accum_sh, out_hbm, cid, sid)


BNM = 1000

def _mix_body(q_ref, mu_ref, pd_ref, pA_ref, p0_ref, p1_ref, p2_ref,
              Wv_ref, Wm1_ref, bm1_ref, Wm2_ref, bm2_ref, qo_ref, muo_ref):
    q = q_ref[...]
    mu = mu_ref[...]
    deg = jnp.maximum(pd_ref[0, :, :1] + pd_ref[1, :, :1], 1.0)
    q1 = q + (pA_ref[0] + pA_ref[1]) / deg
    vm = jnp.stack([p0_ref[0] + p0_ref[1],
                    p1_ref[0] + p1_ref[1],
                    p2_ref[0] + p2_ref[1]], axis=1)
    mu1 = mu + vm / deg[:, :, None]
    mc = jnp.dot(mu1.reshape(BNM * 3, H), Wv_ref[...],
                 preferred_element_type=jnp.float32).reshape(BNM, 3, 2 * H)
    mu_v = mc[..., :H]
    mu_w = mc[..., H:]
    mu_v_norm = jnp.sqrt(jnp.sum(mu_v * mu_v, axis=1) + 1e-8)
    si = jnp.concatenate([q1, mu_v_norm], axis=-1)
    h = jnp.dot(si, Wm1_ref[...],
                preferred_element_type=jnp.float32) + bm1_ref[...]
    h = _silu(h)
    delta = jnp.dot(h, Wm2_ref[...],
                    preferred_element_type=jnp.float32) + bm2_ref[...]
    dq = delta[:, :H]
    dsc = delta[:, H:2 * H]
    dqmu = delta[:, 2 * H:]
    inner = jnp.sum(mu_v * mu_w, axis=1)
    qo_ref[...] = q1 + dq + dqmu * inner
    muo_ref[...] = mu1 + mu_w * dsc[:, None, :]


def _mix_call(q, mu, pd, pA, p0, p1, p2, Wv, Wm1, bm1, Wm2, bm2):
    grid = (N // BNM,)
    nb = lambda i: (i, 0)
    wb = lambda i: (0, 0)
    nb3 = lambda i: (i, 0, 0)
    pb = lambda i: (0, i, 0)
    return pl.pallas_call(
        _mix_body,
        grid=grid,
        in_specs=[
            pl.BlockSpec((BNM, H), nb),
            pl.BlockSpec((BNM, 3, H), nb3),
            pl.BlockSpec((NC, BNM, H), pb),
            pl.BlockSpec((NC, BNM, H), pb),
            pl.BlockSpec((NC, BNM, H), pb),
            pl.BlockSpec((NC, BNM, H), pb),
            pl.BlockSpec((NC, BNM, H), pb),
            pl.BlockSpec((H, 2 * H), wb),
            pl.BlockSpec((2 * H, 3 * H), wb),
            pl.BlockSpec((3 * H,), lambda i: (0,)),
            pl.BlockSpec((3 * H, 3 * H), wb),
            pl.BlockSpec((3 * H,), lambda i: (0,)),
        ],
        out_specs=[
            pl.BlockSpec((BNM, H), nb),
            pl.BlockSpec((BNM, 3, H), nb3),
        ],
        out_shape=[
            jax.ShapeDtypeStruct((N, H), jnp.float32),
            jax.ShapeDtypeStruct((N, 3, H), jnp.float32),
        ],
    )(q, mu, pd, pA, p0, p1, p2, Wv, Wm1, bm1, Wm2, bm2)


def kernel(q, mu, edge_index, rbf, unit_vectors, cutoff_values,
           W1, b1, W2, b2, Wf1, bf1, Wf2, bf2, Wv, Wm1, bm1, Wm2, bm2):
    src = edge_index[1]
    tgt = edge_index[0]
    cut2 = cutoff_values[:, None]
    pd = _sc_deg(tgt)
    fq, fr0, fr1, fr2, fmu = _filters_call(rbf, cut2, unit_vectors,
                                           Wf1, bf1, Wf2, bf2)
    xq, xr, xm = _nodemlp_call(q, W1, b1, W2, b2)
    mu0 = mu[:, 0]
    mu1t = mu[:, 1]
    mu2t = mu[:, 2]
    pA = _sc_scalar(src, tgt, xq, fq)
    p0 = _sc_vec(src, tgt, xr, xm, mu0, fr0, fmu)
    p1 = _sc_vec(src, tgt, xr, xm, mu1t, fr1, fmu)
    p2 = _sc_vec(src, tgt, xr, xm, mu2t, fr2, fmu)
    return _mix_call(q, mu, pd, pA, p0, p1, p2, Wv, Wm1, bm1, Wm2, bm2)
